# baseline (speedup 1.0000x reference)
import functools

import jax
import jax.numpy as jnp
from jax import lax
from jax.experimental import pallas as pl
from jax.experimental.pallas import tpu as pltpu
from jax.experimental.pallas import tpu_sc as plsc

N_CLS = 1024
CTX_LEN = 77
VOCAB = 49408
CTX_DIM = 512

B = N_CLS * CTX_LEN
NW = 32
ROWS_PER_W = B // NW
CHUNK = 112
NCHUNK = ROWS_PER_W // CHUNK

_mesh = plsc.VectorSubcoreMesh(core_axis_name="c", subcore_axis_name="s")


@functools.partial(
    pl.kernel,
    out_type=jax.ShapeDtypeStruct((B, CTX_DIM), jnp.float32),
    mesh=_mesh,
    scratch_types=[
        pltpu.VMEM((ROWS_PER_W,), jnp.int32),
        pltpu.VMEM((CHUNK, CTX_DIM), jnp.float32),
        pltpu.VMEM((CHUNK, CTX_DIM), jnp.float32),
        pltpu.SemaphoreType.DMA,
        pltpu.SemaphoreType.DMA,
        pltpu.SemaphoreType.DMA,
        pltpu.SemaphoreType.DMA,
    ],
)
def _gather(idx_hbm, table_hbm, out_hbm, idx_v, buf0, buf1, sg0, sg1, sw0, sw1):
    wid = lax.axis_index("s") * 2 + lax.axis_index("c")
    base = pl.multiple_of(wid * ROWS_PER_W, ROWS_PER_W)

    pltpu.sync_copy(idx_hbm.at[pl.ds(base, ROWS_PER_W)], idx_v)

    def start_gather(j, buf, sem):
        off = pl.multiple_of(j * CHUNK, CHUNK)
        return pltpu.async_copy(table_hbm.at[idx_v.at[pl.ds(off, CHUNK)]], buf, sem)

    def wait_gather(buf, sem):
        pltpu.make_async_copy(table_hbm.at[idx_v.at[pl.ds(0, CHUNK)]], buf, sem).wait()

    def start_write(j, buf, sem):
        off = pl.multiple_of(base + j * CHUNK, CHUNK)
        return pltpu.async_copy(buf, out_hbm.at[pl.ds(off, CHUNK)], sem)

    def wait_write(buf, sem):
        pltpu.make_async_copy(buf, out_hbm.at[pl.ds(base, CHUNK)], sem).wait()

    start_gather(0, buf0, sg0)
    start_gather(1, buf1, sg1)

    @pl.loop(0, (NCHUNK - 2) // 2)
    def _steady(i):
        j = i * 2
        wait_gather(buf0, sg0)
        start_write(j, buf0, sw0)
        wait_gather(buf1, sg1)
        start_write(j + 1, buf1, sw1)
        wait_write(buf0, sw0)
        start_gather(j + 2, buf0, sg0)
        wait_write(buf1, sw1)
        start_gather(j + 3, buf1, sg1)

    wait_gather(buf0, sg0)
    start_write(NCHUNK - 2, buf0, sw0)
    wait_gather(buf1, sg1)
    start_write(NCHUNK - 1, buf1, sw1)
    wait_write(buf0, sw0)
    wait_write(buf1, sw1)


def kernel(tokenized_prompts, token_embedding):
    idx = tokenized_prompts.reshape(B)
    out = _gather(idx, token_embedding)
    return out.reshape(N_CLS, CTX_LEN, CTX_DIM)


# speedup vs baseline: 2.7052x; 2.7052x over previous
import functools

import jax
import jax.numpy as jnp
from jax import lax
from jax.experimental import pallas as pl
from jax.experimental.pallas import tpu as pltpu
from jax.experimental.pallas import tpu_sc as plsc

N_CLS = 1024
CTX_LEN = 77
VOCAB = 49408
CTX_DIM = 512

NW = 32
CLS_PER_W = N_CLS // NW
IDS_PER_W = CTX_LEN * CLS_PER_W

_mesh = plsc.VectorSubcoreMesh(core_axis_name="c", subcore_axis_name="s")


@functools.partial(
    pl.kernel,
    out_type=jax.ShapeDtypeStruct((CTX_LEN, N_CLS, CTX_DIM), jnp.float32),
    mesh=_mesh,
    scratch_types=[
        pltpu.VMEM((IDS_PER_W,), jnp.int32),
        pltpu.VMEM((CLS_PER_W, CTX_DIM), jnp.float32),
        pltpu.VMEM((CLS_PER_W, CTX_DIM), jnp.float32),
        pltpu.SemaphoreType.DMA,
        pltpu.SemaphoreType.DMA,
        pltpu.SemaphoreType.DMA,
        pltpu.SemaphoreType.DMA,
    ],
)
def _gather(idx_hbm, table_hbm, out_hbm, idx_v, buf0, buf1, sg0, sg1, sw0, sw1):
    wid = lax.axis_index("s") * 2 + lax.axis_index("c")
    base = pl.multiple_of(wid * CLS_PER_W, CLS_PER_W)

    pltpu.sync_copy(idx_hbm.at[pl.ds(wid * IDS_PER_W, IDS_PER_W)], idx_v)

    def idx_slice(t):
        off = pl.multiple_of(t * CLS_PER_W, CLS_PER_W)
        return idx_v.at[pl.ds(off, CLS_PER_W)]

    def start_gather(t, buf, sem):
        return pltpu.async_copy(table_hbm.at[idx_slice(t)], buf, sem)

    def wait_gather(buf, sem):
        pltpu.make_async_copy(table_hbm.at[idx_slice(0)], buf, sem).wait()

    def start_write(t, buf, sem):
        return pltpu.async_copy(buf, out_hbm.at[t, pl.ds(base, CLS_PER_W)], sem)

    def wait_write(buf, sem):
        pltpu.make_async_copy(buf, out_hbm.at[0, pl.ds(base, CLS_PER_W)], sem).wait()

    start_gather(0, buf0, sg0)
    start_gather(1, buf1, sg1)

    @pl.loop(0, (CTX_LEN - 3) // 2)
    def _steady(i):
        t = i * 2
        wait_gather(buf0, sg0)
        start_write(t, buf0, sw0)
        wait_gather(buf1, sg1)
        start_write(t + 1, buf1, sw1)
        wait_write(buf0, sw0)
        start_gather(t + 2, buf0, sg0)
        wait_write(buf1, sw1)
        start_gather(t + 3, buf1, sg1)

    wait_gather(buf0, sg0)
    start_write(CTX_LEN - 3, buf0, sw0)
    wait_gather(buf1, sg1)
    start_write(CTX_LEN - 2, buf1, sw1)
    wait_write(buf0, sw0)
    start_gather(CTX_LEN - 1, buf0, sg0)
    wait_gather(buf0, sg0)
    start_write(CTX_LEN - 1, buf0, sw0)
    wait_write(buf1, sw1)
    wait_write(buf0, sw0)


def kernel(tokenized_prompts, token_embedding):
    idx = (tokenized_prompts.T.reshape(CTX_LEN, NW, CLS_PER_W)
           .transpose(1, 0, 2).reshape(NW * IDS_PER_W))
    out = _gather(idx, token_embedding)
    return out.transpose(1, 0, 2)


# speedup vs baseline: 2.8814x; 1.0651x over previous
import functools

import jax
import jax.numpy as jnp
from jax import lax
from jax.experimental import pallas as pl
from jax.experimental.pallas import tpu as pltpu
from jax.experimental.pallas import tpu_sc as plsc

N_CLS = 1024
CTX_LEN = 77
VOCAB = 49408
CTX_DIM = 512

NW = 32
CLS_PER_W = N_CLS // NW
IDS_PER_W = CTX_LEN * CLS_PER_W

SLAB = 3
NSLAB = 26
LAST_START = CTX_LEN - SLAB

_mesh = plsc.VectorSubcoreMesh(core_axis_name="c", subcore_axis_name="s")


@functools.partial(
    pl.kernel,
    out_type=jax.ShapeDtypeStruct((CTX_LEN, N_CLS, CTX_DIM), jnp.float32),
    mesh=_mesh,
    scratch_types=[
        pltpu.VMEM((IDS_PER_W,), jnp.int32),
        pltpu.VMEM((SLAB, CLS_PER_W, CTX_DIM), jnp.float32),
        pltpu.VMEM((SLAB, CLS_PER_W, CTX_DIM), jnp.float32),
        pltpu.SemaphoreType.DMA,
        pltpu.SemaphoreType.DMA,
        pltpu.SemaphoreType.DMA,
        pltpu.SemaphoreType.DMA,
    ],
)
def _gather(idx_hbm, table_hbm, out_hbm, idx_v, buf0, buf1, sg0, sg1, sw0, sw1):
    wid = lax.axis_index("s") * 2 + lax.axis_index("c")
    base = pl.multiple_of(wid * CLS_PER_W, CLS_PER_W)

    pltpu.sync_copy(idx_hbm.at[pl.ds(wid * IDS_PER_W, IDS_PER_W)], idx_v)

    def slab_t0(s):
        return jnp.minimum(s * SLAB, LAST_START)

    def start_gathers(s, buf, sem):
        t0 = slab_t0(s)
        for j in range(SLAB):
            off = pl.multiple_of((t0 + j) * CLS_PER_W, CLS_PER_W)
            pltpu.async_copy(
                table_hbm.at[idx_v.at[pl.ds(off, CLS_PER_W)]], buf.at[j], sem)

    def wait_gathers(buf, sem):
        for j in range(SLAB):
            pltpu.make_async_copy(
                table_hbm.at[idx_v.at[pl.ds(0, CLS_PER_W)]], buf.at[j], sem).wait()

    def start_write(s, buf, sem):
        t0 = slab_t0(s)
        return pltpu.async_copy(
            buf, out_hbm.at[pl.ds(t0, SLAB), pl.ds(base, CLS_PER_W)], sem)

    def wait_write(buf, sem):
        pltpu.make_async_copy(
            buf, out_hbm.at[pl.ds(0, SLAB), pl.ds(base, CLS_PER_W)], sem).wait()

    start_gathers(0, buf0, sg0)
    start_gathers(1, buf1, sg1)

    @pl.loop(0, (NSLAB - 2) // 2)
    def _steady(i):
        s = i * 2
        wait_gathers(buf0, sg0)
        start_write(s, buf0, sw0)
        wait_gathers(buf1, sg1)
        start_write(s + 1, buf1, sw1)
        wait_write(buf0, sw0)
        start_gathers(s + 2, buf0, sg0)
        wait_write(buf1, sw1)
        start_gathers(s + 3, buf1, sg1)

    wait_gathers(buf0, sg0)
    start_write(NSLAB - 2, buf0, sw0)
    wait_gathers(buf1, sg1)
    start_write(NSLAB - 1, buf1, sw1)
    wait_write(buf0, sw0)
    wait_write(buf1, sw1)


def kernel(tokenized_prompts, token_embedding):
    idx = (tokenized_prompts.T.reshape(CTX_LEN, NW, CLS_PER_W)
           .transpose(1, 0, 2).reshape(NW * IDS_PER_W))
    out = _gather(idx, token_embedding)
    return out.transpose(1, 0, 2)


# speedup vs baseline: 2.9200x; 1.0134x over previous
import functools

import jax
import jax.numpy as jnp
from jax import lax
from jax.experimental import pallas as pl
from jax.experimental.pallas import tpu as pltpu
from jax.experimental.pallas import tpu_sc as plsc

N_CLS = 1024
CTX_LEN = 77
VOCAB = 49408
CTX_DIM = 512

NW = 32
CLS_PER_W = N_CLS // NW
IDS_PER_W = CTX_LEN * CLS_PER_W

SLAB_A = 4
SLAB_B = 3
NPAIR = CTX_LEN // (SLAB_A + SLAB_B)

_mesh = plsc.VectorSubcoreMesh(core_axis_name="c", subcore_axis_name="s")


@functools.partial(
    pl.kernel,
    out_type=jax.ShapeDtypeStruct((CTX_LEN, N_CLS, CTX_DIM), jnp.float32),
    mesh=_mesh,
    scratch_types=[
        pltpu.VMEM((IDS_PER_W,), jnp.int32),
        pltpu.VMEM((SLAB_A, CLS_PER_W, CTX_DIM), jnp.float32),
        pltpu.VMEM((SLAB_B, CLS_PER_W, CTX_DIM), jnp.float32),
        pltpu.SemaphoreType.DMA,
        pltpu.SemaphoreType.DMA,
        pltpu.SemaphoreType.DMA,
        pltpu.SemaphoreType.DMA,
    ],
)
def _gather(idx_hbm, table_hbm, out_hbm, idx_v, buf0, buf1, sg0, sg1, sw0, sw1):
    wid = lax.axis_index("s") * 2 + lax.axis_index("c")
    base = pl.multiple_of(wid * CLS_PER_W, CLS_PER_W)

    pltpu.sync_copy(idx_hbm.at[pl.ds(wid * IDS_PER_W, IDS_PER_W)], idx_v)

    def start_gathers(t0, n, buf, sem):
        for j in range(n):
            off = pl.multiple_of((t0 + j) * CLS_PER_W, CLS_PER_W)
            pltpu.async_copy(
                table_hbm.at[idx_v.at[pl.ds(off, CLS_PER_W)]], buf.at[j], sem)

    def wait_gathers(n, buf, sem):
        for j in range(n):
            pltpu.make_async_copy(
                table_hbm.at[idx_v.at[pl.ds(0, CLS_PER_W)]], buf.at[j], sem).wait()

    def start_write(t0, n, buf, sem):
        return pltpu.async_copy(
            buf, out_hbm.at[pl.ds(t0, n), pl.ds(base, CLS_PER_W)], sem)

    def wait_write(n, buf, sem):
        pltpu.make_async_copy(
            buf, out_hbm.at[pl.ds(0, n), pl.ds(base, CLS_PER_W)], sem).wait()

    start_gathers(0, SLAB_A, buf0, sg0)
    start_gathers(SLAB_A, SLAB_B, buf1, sg1)

    @pl.loop(0, NPAIR - 1)
    def _steady(i):
        t0 = i * (SLAB_A + SLAB_B)
        nxt = t0 + SLAB_A + SLAB_B
        wait_gathers(SLAB_A, buf0, sg0)
        start_write(t0, SLAB_A, buf0, sw0)
        wait_gathers(SLAB_B, buf1, sg1)
        start_write(t0 + SLAB_A, SLAB_B, buf1, sw1)
        wait_write(SLAB_A, buf0, sw0)
        start_gathers(nxt, SLAB_A, buf0, sg0)
        wait_write(SLAB_B, buf1, sw1)
        start_gathers(nxt + SLAB_A, SLAB_B, buf1, sg1)

    last = (NPAIR - 1) * (SLAB_A + SLAB_B)
    wait_gathers(SLAB_A, buf0, sg0)
    start_write(last, SLAB_A, buf0, sw0)
    wait_gathers(SLAB_B, buf1, sg1)
    start_write(last + SLAB_A, SLAB_B, buf1, sw1)
    wait_write(SLAB_A, buf0, sw0)
    wait_write(SLAB_B, buf1, sw1)


def kernel(tokenized_prompts, token_embedding):
    idx = (tokenized_prompts.T.reshape(CTX_LEN, NW, CLS_PER_W)
           .transpose(1, 0, 2).reshape(NW * IDS_PER_W))
    out = _gather(idx, token_embedding)
    return out.transpose(1, 0, 2)


# speedup vs baseline: 2.9962x; 1.0261x over previous
import functools

import jax
import jax.numpy as jnp
from jax import lax
from jax.experimental import pallas as pl
from jax.experimental.pallas import tpu as pltpu
from jax.experimental.pallas import tpu_sc as plsc

N_CLS = 1024
CTX_LEN = 77
VOCAB = 49408
CTX_DIM = 512

NW = 32
CLS_PER_W = N_CLS // NW
IDS_PER_W = CTX_LEN * CLS_PER_W

SLABS = (3, 2, 2)
CYCLE = sum(SLABS)
NCYC = CTX_LEN // CYCLE
OFFS = (0, 3, 5)

_mesh = plsc.VectorSubcoreMesh(core_axis_name="c", subcore_axis_name="s")


@functools.partial(
    pl.kernel,
    out_type=jax.ShapeDtypeStruct((CTX_LEN, N_CLS, CTX_DIM), jnp.float32),
    mesh=_mesh,
    scratch_types=[
        pltpu.VMEM((IDS_PER_W,), jnp.int32),
        pltpu.VMEM((SLABS[0], CLS_PER_W, CTX_DIM), jnp.float32),
        pltpu.VMEM((SLABS[1], CLS_PER_W, CTX_DIM), jnp.float32),
        pltpu.VMEM((SLABS[2], CLS_PER_W, CTX_DIM), jnp.float32),
        pltpu.SemaphoreType.DMA,
        pltpu.SemaphoreType.DMA,
        pltpu.SemaphoreType.DMA,
        pltpu.SemaphoreType.DMA,
        pltpu.SemaphoreType.DMA,
        pltpu.SemaphoreType.DMA,
    ],
)
def _gather(idx_hbm, table_hbm, out_hbm, idx_v,
            buf0, buf1, buf2, sg0, sg1, sg2, sw0, sw1, sw2):
    wid = lax.axis_index("s") * 2 + lax.axis_index("c")
    base = pl.multiple_of(wid * CLS_PER_W, CLS_PER_W)

    pltpu.sync_copy(idx_hbm.at[pl.ds(wid * IDS_PER_W, IDS_PER_W)], idx_v)

    def start_gathers(t0, n, buf, sem):
        for j in range(n):
            off = pl.multiple_of((t0 + j) * CLS_PER_W, CLS_PER_W)
            pltpu.async_copy(
                table_hbm.at[idx_v.at[pl.ds(off, CLS_PER_W)]], buf.at[j], sem)

    def wait_gathers(n, buf, sem):
        for j in range(n):
            pltpu.make_async_copy(
                table_hbm.at[idx_v.at[pl.ds(0, CLS_PER_W)]], buf.at[j], sem).wait()

    def start_write(t0, n, buf, sem):
        return pltpu.async_copy(
            buf, out_hbm.at[pl.ds(t0, n), pl.ds(base, CLS_PER_W)], sem)

    def wait_write(n, buf, sem):
        pltpu.make_async_copy(
            buf, out_hbm.at[pl.ds(0, n), pl.ds(base, CLS_PER_W)], sem).wait()

    lanes = ((buf0, sg0, sw0), (buf1, sg1, sw1), (buf2, sg2, sw2))

    for k in range(3):
        buf, sg, _ = lanes[k]
        start_gathers(OFFS[k], SLABS[k], buf, sg)

    @pl.loop(0, NCYC - 1)
    def _steady(i):
        t0 = i * CYCLE
        for k in range(3):
            buf, sg, sw = lanes[k]
            wait_gathers(SLABS[k], buf, sg)
            start_write(t0 + OFFS[k], SLABS[k], buf, sw)
            wait_write(SLABS[k], buf, sw)
            start_gathers(t0 + CYCLE + OFFS[k], SLABS[k], buf, sg)

    last = (NCYC - 1) * CYCLE
    for k in range(3):
        buf, sg, sw = lanes[k]
        wait_gathers(SLABS[k], buf, sg)
        start_write(last + OFFS[k], SLABS[k], buf, sw)
    for k in range(3):
        buf, _, sw = lanes[k]
        wait_write(SLABS[k], buf, sw)


def kernel(tokenized_prompts, token_embedding):
    idx = (tokenized_prompts.T.reshape(CTX_LEN, NW, CLS_PER_W)
           .transpose(1, 0, 2).reshape(NW * IDS_PER_W))
    out = _gather(idx, token_embedding)
    return out.transpose(1, 0, 2)
